# baseline (speedup 1.0000x reference)
import functools

import jax
import jax.numpy as jnp
from jax import lax
from jax.experimental import pallas as pl
from jax.experimental.pallas import tpu as pltpu
from jax.experimental.pallas import tpu_sc as plsc

B = 16
C = 384
CH = C // 2
HW = 4096
P = 256
NC, NS = 2, 16
NW = NC * NS
RPT = B * P // NW
L = 16
NV = CH // L
OPAD = 2 * 128


def _rsqrt(s):
    i = plsc.bitcast(s, jnp.int32)
    i = jnp.int32(0x5F3759DF) - lax.shift_right_arithmetic(i, 1)
    y = plsc.bitcast(i, jnp.float32)
    half = s * 0.5
    for _ in range(3):
        y = y * (1.5 - half * y * y)
    return y


def _sc_body(x_hbm, pid_hbm, out_hbm, pid_v, idx_v, rows_v, ssq_all,
             fac_all, out_local, sem):
    cid = lax.axis_index("c")
    sid = lax.axis_index("s")
    wid = cid * NS + sid
    b = lax.div(wid, 2)
    poff = lax.rem(wid, 2) * RPT

    with jax.named_scope("idx_setup"):
        pltpu.sync_copy(pid_hbm, pid_v)

        base = b * HW
        for k in range(RPT // L):
            pv = pid_v[pl.ds(poff + k * L, L)]
            idx_v[pl.ds(k * L, L)] = pv + base

    with jax.named_scope("row_gather"):
        pltpu.async_copy(x_hbm.at[idx_v], rows_v, sem).wait()

    iota = lax.iota(jnp.int32, L)

    def row_ssq(r, _):
        v = rows_v[r, pl.ds(0, L)]
        acc = v * v
        for t in range(1, NV):
            v = rows_v[r, pl.ds(t * L, L)]
            acc = acc + v * v
        ssq_all[r] = plsc.cumsum(acc)
        return 0

    with jax.named_scope("ssq"):
        lax.fori_loop(0, RPT, row_ssq, 0)

    with jax.named_scope("newton"):
        lane15 = jnp.full((L,), L - 1, dtype=jnp.int32)
        for g in range(RPT // L):
            sg = plsc.load_gather(ssq_all, [iota + g * L, lane15])
            norm = sg * _rsqrt(sg)
            fac_all[pl.ds(g * L, L)] = 1.0 / (norm + 1e-7)

    def chan_scale(c2, _):
        for u in range(2):
            c = c2 * 2 + u
            cv = jnp.full((L,), c, dtype=jnp.int32)
            for g in range(RPT // L):
                prow = iota + g * L
                fv = fac_all[pl.ds(g * L, L)]
                vals = plsc.load_gather(rows_v, [prow, cv])
                out_local[c, pl.ds(g * L, L)] = vals * fv
        return 0

    with jax.named_scope("scale"):
        lax.fori_loop(0, CH // 2, chan_scale, 0)

    with jax.named_scope("writeout"):
        pltpu.sync_copy(out_local,
                        out_hbm.at[pl.ds(0, CH), pl.ds(wid * RPT, RPT)])


@jax.jit
def _run(x4, patch_ids):
    xt = jnp.transpose(x4, (0, 2, 3, 1)).reshape(B * HW, C)
    mesh = plsc.VectorSubcoreMesh(
        core_axis_name="c", subcore_axis_name="s",
        num_cores=NC, num_subcores=NS)
    f = pl.kernel(
        _sc_body,
        out_type=jax.ShapeDtypeStruct((CH, B * P), jnp.float32),
        mesh=mesh,
        scratch_types=[
            pltpu.VMEM((P,), jnp.int32),
            pltpu.VMEM((RPT,), jnp.int32),
            pltpu.VMEM((RPT, C), jnp.float32),
            pltpu.VMEM((RPT, L), jnp.float32),
            pltpu.VMEM((RPT,), jnp.float32),
            pltpu.VMEM((CH, RPT), jnp.float32),
            pltpu.SemaphoreType.DMA,
        ],
        compiler_params=pltpu.CompilerParams(
            use_tc_tiling_on_sc=True, needs_layout_passes=False),
    )
    return jnp.transpose(f(xt, patch_ids))


def kernel(x, num_patches, patch_ids):
    out = _run(x, patch_ids)
    return (out, patch_ids)


# speedup vs baseline: 1.2041x; 1.2041x over previous
import functools

import jax
import jax.numpy as jnp
from jax import lax
from jax.experimental import pallas as pl
from jax.experimental.pallas import tpu as pltpu
from jax.experimental.pallas import tpu_sc as plsc

B = 16
C = 384
CH = C // 2
HW = 4096
P = 256
NC, NS = 2, 16
NW = NC * NS
RPT = B * P // NW
L = 16
NV = CH // L
OPAD = 2 * 128


def _rsqrt(s):
    i = plsc.bitcast(s, jnp.int32)
    i = jnp.int32(0x5F3759DF) - lax.shift_right_arithmetic(i, 1)
    y = plsc.bitcast(i, jnp.float32)
    half = s * 0.5
    for _ in range(3):
        y = y * (1.5 - half * y * y)
    return y


def _sc_body(x_hbm, pid_hbm, out_hbm, pid_v, idx_v, rows_v, ssq_all,
             fac_all, out_local, sem):
    cid = lax.axis_index("c")
    sid = lax.axis_index("s")
    wid = cid * NS + sid
    b = lax.div(wid, 2)
    poff = lax.rem(wid, 2) * RPT

    with jax.named_scope("idx_setup"):
        pltpu.sync_copy(pid_hbm, pid_v)

        base = b * HW
        for k in range(RPT // L):
            pv = pid_v[pl.ds(poff + k * L, L)]
            idx_v[pl.ds(k * L, L)] = pv + base

    with jax.named_scope("row_gather"):
        pltpu.async_copy(x_hbm.at[idx_v], rows_v, sem).wait()

    iota = lax.iota(jnp.int32, L)

    def row_ssq(r, _):
        v = rows_v[r, pl.ds(0, L)]
        acc = v * v
        for t in range(1, NV):
            v = rows_v[r, pl.ds(t * L, L)]
            acc = acc + v * v
        ssq_all[r] = plsc.cumsum(acc)
        return 0

    with jax.named_scope("ssq"):
        lax.fori_loop(0, RPT, row_ssq, 0)

    with jax.named_scope("newton"):
        lane15 = jnp.full((L,), L - 1, dtype=jnp.int32)
        for g in range(RPT // L):
            sg = plsc.load_gather(ssq_all, [iota + g * L, lane15])
            norm = sg * _rsqrt(sg)
            fac_all[pl.ds(g * L, L)] = 1.0 / (norm + 1e-7)

    def row_scale(r, _):
        fv = plsc.load_gather(fac_all, [jnp.full((L,), r, dtype=jnp.int32)])
        for t in range(NV):
            out_local[r, pl.ds(t * L, L)] = rows_v[r, pl.ds(t * L, L)] * fv
        return 0

    with jax.named_scope("scale"):
        lax.fori_loop(0, RPT, row_scale, 0)

    with jax.named_scope("writeout"):
        pltpu.sync_copy(out_local, out_hbm.at[pl.ds(wid * RPT, RPT)])


@jax.jit
def _run(x4, patch_ids):
    xt = jnp.transpose(x4, (0, 2, 3, 1)).reshape(B * HW, C)
    mesh = plsc.VectorSubcoreMesh(
        core_axis_name="c", subcore_axis_name="s",
        num_cores=NC, num_subcores=NS)
    f = pl.kernel(
        _sc_body,
        out_type=jax.ShapeDtypeStruct((B * P, OPAD), jnp.float32),
        mesh=mesh,
        scratch_types=[
            pltpu.VMEM((P,), jnp.int32),
            pltpu.VMEM((RPT,), jnp.int32),
            pltpu.VMEM((RPT, C), jnp.float32),
            pltpu.VMEM((RPT, L), jnp.float32),
            pltpu.VMEM((RPT,), jnp.float32),
            pltpu.VMEM((RPT, OPAD), jnp.float32),
            pltpu.SemaphoreType.DMA,
        ],
        compiler_params=pltpu.CompilerParams(
            use_tc_tiling_on_sc=True, needs_layout_passes=False),
    )
    return f(xt, patch_ids)[:, :CH]


def kernel(x, num_patches, patch_ids):
    out = _run(x, patch_ids)
    return (out, patch_ids)


# speedup vs baseline: 1.3193x; 1.0957x over previous
import functools

import jax
import jax.numpy as jnp
from jax import lax
from jax.experimental import pallas as pl
from jax.experimental.pallas import tpu as pltpu
from jax.experimental.pallas import tpu_sc as plsc

B = 16
C = 384
CH = C // 2
HW = 4096
P = 256
NC, NS = 2, 16
NW = NC * NS
RPT = B * P // NW
L = 16
NV = CH // L
OPAD = 2 * 128


def _rsqrt(s):
    i = plsc.bitcast(s, jnp.int32)
    i = jnp.int32(0x5F3759DF) - lax.shift_right_arithmetic(i, 1)
    y = plsc.bitcast(i, jnp.float32)
    half = s * 0.5
    for _ in range(3):
        y = y * (1.5 - half * y * y)
    return y


def _sc_body(x_hbm, pid_hbm, out_hbm, pid_v, idx_v, idx0, idx1, rows_v,
             ssq_all, out_local, sem, sem1, semw):
    cid = lax.axis_index("c")
    sid = lax.axis_index("s")
    wid = cid * NS + sid
    b = lax.div(wid, 2)
    poff = lax.rem(wid, 2) * RPT

    RH = RPT // 2

    with jax.named_scope("idx_setup"):
        pltpu.sync_copy(pid_hbm, pid_v)

        base = b * HW
        for k in range(RPT // L):
            pv = pid_v[pl.ds(poff + k * L, L)]
            idx_v[pl.ds(k * L, L)] = pv + base
        for k in range(RH // L):
            idx0[pl.ds(k * L, L)] = idx_v[pl.ds(k * L, L)]
            idx1[pl.ds(k * L, L)] = idx_v[pl.ds(RH + k * L, L)]

    with jax.named_scope("row_gather_start"):
        d0 = pltpu.async_copy(x_hbm.at[idx0], rows_v.at[pl.ds(0, RH)], sem)
        d1 = pltpu.async_copy(x_hbm.at[idx1], rows_v.at[pl.ds(RH, RH)],
                              sem1)

    lane15 = jnp.full((L,), L - 1, dtype=jnp.int32)

    def row_norm(i, base_r):
        r = base_r + i
        v = [rows_v[r, pl.ds(t * L, L)] for t in range(NV)]
        acc = v[0] * v[0]
        for t in range(1, NV):
            acc = acc + v[t] * v[t]
        ssq_all[r] = plsc.cumsum(acc)
        rv = jnp.full((L,), r, dtype=jnp.int32)
        s = plsc.load_gather(ssq_all, [rv, lane15])
        norm = s * _rsqrt(s)
        fac = 1.0 / (norm + 1e-7)
        for t in range(NV):
            out_local[r, pl.ds(t * L, L)] = v[t] * fac
        return base_r

    with jax.named_scope("half0"):
        d0.wait()
        lax.fori_loop(0, RH, row_norm, 0)
        w0 = pltpu.async_copy(
            out_local.at[pl.ds(0, RH)],
            out_hbm.at[pl.ds(wid * RPT, RH)], semw)

    with jax.named_scope("half1"):
        d1.wait()
        lax.fori_loop(0, RH, row_norm, RH)
        pltpu.sync_copy(out_local.at[pl.ds(RH, RH)],
                        out_hbm.at[pl.ds(wid * RPT + RH, RH)])
        w0.wait()


@jax.jit
def _run(x4, patch_ids):
    xt = jnp.transpose(x4, (0, 2, 3, 1)).reshape(B * HW, C)
    mesh = plsc.VectorSubcoreMesh(
        core_axis_name="c", subcore_axis_name="s",
        num_cores=NC, num_subcores=NS)
    f = pl.kernel(
        _sc_body,
        out_type=jax.ShapeDtypeStruct((B * P, OPAD), jnp.float32),
        mesh=mesh,
        scratch_types=[
            pltpu.VMEM((P,), jnp.int32),
            pltpu.VMEM((RPT,), jnp.int32),
            pltpu.VMEM((RPT // 2,), jnp.int32),
            pltpu.VMEM((RPT // 2,), jnp.int32),
            pltpu.VMEM((RPT, C), jnp.float32),
            pltpu.VMEM((RPT, L), jnp.float32),
            pltpu.VMEM((RPT, OPAD), jnp.float32),
            pltpu.SemaphoreType.DMA,
            pltpu.SemaphoreType.DMA,
            pltpu.SemaphoreType.DMA,
        ],
        compiler_params=pltpu.CompilerParams(
            use_tc_tiling_on_sc=True, needs_layout_passes=False),
    )
    return f(xt, patch_ids)[:, :CH]


def kernel(x, num_patches, patch_ids):
    out = _run(x, patch_ids)
    return (out, patch_ids)


# speedup vs baseline: 1.3860x; 1.0505x over previous
import functools

import jax
import jax.numpy as jnp
from jax import lax
from jax.experimental import pallas as pl
from jax.experimental.pallas import tpu as pltpu
from jax.experimental.pallas import tpu_sc as plsc

B = 16
C = 384
CH = C // 2
HW = 4096
P = 256
NC, NS = 2, 16
NW = NC * NS
RPT = B * P // NW
L = 16
NV = CH // L
OPAD = 2 * 128


def _rsqrt(s):
    i = plsc.bitcast(s, jnp.int32)
    i = jnp.int32(0x5F3759DF) - lax.shift_right_arithmetic(i, 1)
    y = plsc.bitcast(i, jnp.float32)
    half = s * 0.5
    for _ in range(3):
        y = y * (1.5 - half * y * y)
    return y


def _sc_body(x_hbm, pid_hbm, out_hbm, pid_v, idx0, idx1, rows_v,
             ssq_all, out_local, sem, sem1, semw):
    cid = lax.axis_index("c")
    sid = lax.axis_index("s")
    wid = cid * NS + sid
    b = lax.div(wid, 2)
    poff = lax.rem(wid, 2) * RPT

    RH = RPT // 2

    with jax.named_scope("idx_setup"):
        pltpu.sync_copy(pid_hbm.at[pl.ds(poff, RPT)], pid_v)

        base = b * HW
        for k in range(RH // L):
            idx0[pl.ds(k * L, L)] = pid_v[pl.ds(k * L, L)] + base
            idx1[pl.ds(k * L, L)] = pid_v[pl.ds(RH + k * L, L)] + base

    with jax.named_scope("row_gather_start"):
        d0 = pltpu.async_copy(x_hbm.at[idx0], rows_v.at[pl.ds(0, RH)], sem)
        d1 = pltpu.async_copy(x_hbm.at[idx1], rows_v.at[pl.ds(RH, RH)],
                              sem1)

    lane15 = jnp.full((L,), L - 1, dtype=jnp.int32)

    def row_norm(i, base_r):
        rs = [base_r + i * 2, base_r + i * 2 + 1]
        vs, facs = [], []
        for r in rs:
            v = [rows_v[r, pl.ds(t * L, L)] for t in range(NV)]
            acc = v[0] * v[0]
            for t in range(1, NV):
                acc = acc + v[t] * v[t]
            ssq_all[r] = plsc.cumsum(acc)
            vs.append(v)
        for r in rs:
            rv = jnp.full((L,), r, dtype=jnp.int32)
            s = plsc.load_gather(ssq_all, [rv, lane15])
            norm = s * _rsqrt(s)
            facs.append(1.0 / (norm + 1e-7))
        for r, v, fac in zip(rs, vs, facs):
            for t in range(NV):
                out_local[r, pl.ds(t * L, L)] = v[t] * fac
        return base_r

    with jax.named_scope("half0"):
        d0.wait()
        lax.fori_loop(0, RH // 2, row_norm, 0)
        w0 = pltpu.async_copy(
            out_local.at[pl.ds(0, RH)],
            out_hbm.at[pl.ds(wid * RPT, RH)], semw)

    with jax.named_scope("half1"):
        d1.wait()
        lax.fori_loop(0, RH // 2, row_norm, RH)
        pltpu.sync_copy(out_local.at[pl.ds(RH, RH)],
                        out_hbm.at[pl.ds(wid * RPT + RH, RH)])
        w0.wait()


@jax.jit
def _run(x4, patch_ids):
    xt = jnp.transpose(x4, (0, 2, 3, 1)).reshape(B * HW, C)
    mesh = plsc.VectorSubcoreMesh(
        core_axis_name="c", subcore_axis_name="s",
        num_cores=NC, num_subcores=NS)
    f = pl.kernel(
        _sc_body,
        out_type=jax.ShapeDtypeStruct((B * P, OPAD), jnp.float32),
        mesh=mesh,
        scratch_types=[
            pltpu.VMEM((RPT,), jnp.int32),
            pltpu.VMEM((RPT // 2,), jnp.int32),
            pltpu.VMEM((RPT // 2,), jnp.int32),
            pltpu.VMEM((RPT, C), jnp.float32),
            pltpu.VMEM((RPT, L), jnp.float32),
            pltpu.VMEM((RPT, OPAD), jnp.float32),
            pltpu.SemaphoreType.DMA,
            pltpu.SemaphoreType.DMA,
            pltpu.SemaphoreType.DMA,
        ],
        compiler_params=pltpu.CompilerParams(
            use_tc_tiling_on_sc=True, needs_layout_passes=False),
    )
    return f(xt, patch_ids)[:, :CH]


def kernel(x, num_patches, patch_ids):
    out = _run(x, patch_ids)
    return (out, patch_ids)
